# baseline (speedup 1.0000x reference)
import jax
import jax.numpy as jnp
from jax.experimental import pallas as pl
from jax.experimental.pallas import tpu as pltpu

_DEPTH = 8
_H = 8
_LVL_OFF = (0, 8, 16, 24, 32, 48, 80, 144)
_KROWS = 272


def _exp_kernel(k_ref, ek_ref):
    for d in range(_DEPTH):
        n = 1 << d
        off = _LVL_OFF[d]
        blk = k_ref[0, n - 1:2 * n - 1]
        D = blk.shape[-1]
        ek_ref[0, off:off + n, :D] = jnp.exp(-blk[:, 0, :]).astype(jnp.bfloat16)
        ek_ref[0, off:off + n, D:] = jnp.exp(-blk[:, 1, :]).astype(jnp.bfloat16)


def _tree_kernel(q_ref, ek_ref, val_ref, sup_ref, val_out_ref):
    q = q_ref[...]
    TB, D = q.shape
    u = jnp.exp(-q).astype(jnp.bfloat16)
    iu = jnp.exp(q).astype(jnp.bfloat16)
    acc = jnp.zeros((TB, 1), jnp.float32)
    for h in range(_H):
        eks = ek_ref[h]
        ix = jnp.zeros((TB, 1), jnp.int32)
        for d in range(_DEPTH):
            n = 1 << d
            off = _LVL_OFF[d]
            if n == 1:
                g = jnp.broadcast_to(eks[off:off + 1], (TB, 2 * D))
            else:
                iota = jax.lax.broadcasted_iota(jnp.int32, (TB, n), 1)
                oh = (iota == ix).astype(jnp.bfloat16)
                g = jnp.dot(oh, eks[off:off + n],
                            preferred_element_type=jnp.float32
                            ).astype(jnp.bfloat16)
            a = g[:, :D] + u
            b = g[:, D:] + iu
            r = (a * b) / (a + b)
            t = jnp.sum(r, axis=-1, keepdims=True
                        ).astype(jnp.float32)
            bit = t <= 1.0
            ix = 2 * ix + bit.astype(jnp.int32)
            acc = acc + jnp.minimum(t, 1.0 / t)
        iota = jax.lax.broadcasted_iota(jnp.int32, (TB, 1 << _DEPTH), 1)
        oh = (iota == ix).astype(jnp.float32)
        val_out_ref[:, h, :] = jnp.dot(oh, val_ref[h],
                                       preferred_element_type=jnp.float32)
    sup_ref[...] = -jnp.log(acc)


def kernel(query, tree_key, tree_value, random):
    B, D = query.shape
    H = tree_key.shape[0]
    out_dim = tree_value.shape[-1]
    nk = (1 << _DEPTH) - 1
    nv = 1 << _DEPTH
    TB = 256

    ek = pl.pallas_call(
        _exp_kernel,
        grid=(H,),
        in_specs=[pl.BlockSpec((1, nk, 2, D), lambda h: (h, 0, 0, 0))],
        out_specs=pl.BlockSpec((1, _KROWS, 2 * D), lambda h: (h, 0, 0)),
        out_shape=jax.ShapeDtypeStruct((H, _KROWS, 2 * D), jnp.bfloat16),
        compiler_params=pltpu.CompilerParams(
            dimension_semantics=("parallel",)),
    )(tree_key)

    sup, val = pl.pallas_call(
        _tree_kernel,
        grid=(B // TB,),
        in_specs=[
            pl.BlockSpec((TB, D), lambda b: (b, 0)),
            pl.BlockSpec((H, _KROWS, 2 * D), lambda b: (0, 0, 0)),
            pl.BlockSpec((H, nv, out_dim), lambda b: (0, 0, 0)),
        ],
        out_specs=[
            pl.BlockSpec((TB, 1), lambda b: (b, 0)),
            pl.BlockSpec((TB, H, out_dim), lambda b: (b, 0, 0)),
        ],
        out_shape=[
            jax.ShapeDtypeStruct((B, 1), jnp.float32),
            jax.ShapeDtypeStruct((B, H, out_dim), jnp.float32),
        ],
        compiler_params=pltpu.CompilerParams(
            dimension_semantics=("parallel",)),
    )(query, ek, tree_value)
    return sup.reshape(B), val.reshape(B * H, out_dim)


# speedup vs baseline: 1.0104x; 1.0104x over previous
import jax
import jax.numpy as jnp
from jax.experimental import pallas as pl
from jax.experimental.pallas import tpu as pltpu

_DEPTH = 8
_H = 8
_LVL_OFF = (0, 8, 16, 24, 32, 48, 80, 144)
_KROWS = 272


def _tree_kernel(k_ref, q_ref, val_ref, sup_ref, val_out_ref, ek_ref):
    i = pl.program_id(0)

    @pl.when(i < _H)
    def _prologue():
        for d in range(_DEPTH):
            n = 1 << d
            off = _LVL_OFF[d]
            blk = k_ref[0, n - 1:2 * n - 1]
            D = blk.shape[-1]
            ek_ref[i, off:off + n, :D] = (
                jnp.exp(-blk[:, 0, :]).astype(jnp.bfloat16))
            ek_ref[i, off:off + n, D:] = (
                jnp.exp(-blk[:, 1, :]).astype(jnp.bfloat16))

    @pl.when(i >= _H)
    def _main():
        q = q_ref[...]
        TB, D = q.shape
        u = jnp.exp(-q).astype(jnp.bfloat16)
        iu = jnp.exp(q).astype(jnp.bfloat16)
        acc = jnp.zeros((TB, 1), jnp.float32)
        for h in range(_H):
            eks = ek_ref[h]
            ix = jnp.zeros((TB, 1), jnp.int32)
            for d in range(_DEPTH):
                n = 1 << d
                off = _LVL_OFF[d]
                if n == 1:
                    g = jnp.broadcast_to(eks[off:off + 1], (TB, 2 * D))
                else:
                    iota = jax.lax.broadcasted_iota(jnp.int32, (TB, n), 1)
                    oh = (iota == ix).astype(jnp.bfloat16)
                    g = jnp.dot(oh, eks[off:off + n],
                                preferred_element_type=jnp.float32
                                ).astype(jnp.bfloat16)
                a = g[:, :D] + u
                b = g[:, D:] + iu
                r = (a * b) / (a + b)
                t = jnp.sum(r, axis=-1, keepdims=True
                            ).astype(jnp.float32)
                bit = t <= 1.0
                ix = 2 * ix + bit.astype(jnp.int32)
                acc = acc + jnp.minimum(t, 1.0 / t)
            iota = jax.lax.broadcasted_iota(jnp.int32, (TB, 1 << _DEPTH), 1)
            oh = (iota == ix).astype(jnp.float32)
            val_out_ref[:, h, :] = jnp.dot(oh, val_ref[h],
                                           preferred_element_type=jnp.float32)
        sup_ref[...] = -jnp.log(acc)


def kernel(query, tree_key, tree_value, random):
    B, D = query.shape
    H = tree_key.shape[0]
    out_dim = tree_value.shape[-1]
    nk = (1 << _DEPTH) - 1
    nv = 1 << _DEPTH
    TB = 256
    nb = B // TB

    sup, val = pl.pallas_call(
        _tree_kernel,
        grid=(H + nb,),
        in_specs=[
            pl.BlockSpec((1, nk, 2, D),
                         lambda i: (jnp.minimum(i, _H - 1), 0, 0, 0)),
            pl.BlockSpec((TB, D), lambda i: (jnp.maximum(i - _H, 0), 0)),
            pl.BlockSpec((H, nv, out_dim), lambda i: (0, 0, 0)),
        ],
        out_specs=[
            pl.BlockSpec((TB, 1), lambda i: (jnp.maximum(i - _H, 0), 0)),
            pl.BlockSpec((TB, H, out_dim),
                         lambda i: (jnp.maximum(i - _H, 0), 0, 0)),
        ],
        out_shape=[
            jax.ShapeDtypeStruct((B, 1), jnp.float32),
            jax.ShapeDtypeStruct((B, H, out_dim), jnp.float32),
        ],
        scratch_shapes=[pltpu.VMEM((H, _KROWS, 2 * D), jnp.bfloat16)],
    )(tree_key, query, tree_value)
    return sup.reshape(B), val.reshape(B * H, out_dim)


# speedup vs baseline: 1.0359x; 1.0252x over previous
import jax
import jax.numpy as jnp
from jax.experimental import pallas as pl
from jax.experimental.pallas import tpu as pltpu

_DEPTH = 8
_H = 8
_LVL_OFF = (0, 8, 16, 24, 32, 48, 80, 144)
_KROWS = 272


def _tree_kernel(k_ref, q_ref, val_ref, sup_ref, val_out_ref, ek_ref):
    i = pl.program_id(0)

    @pl.when(i < _H)
    def _prologue():
        for d in range(_DEPTH):
            n = 1 << d
            off = _LVL_OFF[d]
            blk = k_ref[0, n - 1:2 * n - 1]
            D = blk.shape[-1]
            ek_ref[i, off:off + n, :D] = (
                jnp.exp(-blk[:, 0, :]).astype(jnp.bfloat16))
            ek_ref[i, off:off + n, D:] = (
                jnp.exp(-blk[:, 1, :]).astype(jnp.bfloat16))

    @pl.when(i >= _H)
    def _main():
        q = q_ref[...]
        TB, D = q.shape
        u = jnp.exp(-q).astype(jnp.bfloat16)
        iu = jnp.exp(q).astype(jnp.bfloat16)
        acc = jnp.zeros((TB, 1), jnp.float32)
        for h in range(_H):
            eks = ek_ref[h]
            ix = jnp.zeros((TB, 1), jnp.int32)
            for d in range(_DEPTH):
                n = 1 << d
                off = _LVL_OFF[d]
                if n == 1:
                    g = jnp.broadcast_to(eks[off:off + 1], (TB, 2 * D))
                elif n == 2:
                    m = (ix == 1).astype(jnp.bfloat16)
                    m0 = (ix == 0).astype(jnp.bfloat16)
                    g = (m0 * eks[off:off + 1]
                         + m * eks[off + 1:off + 2])
                else:
                    iota = jax.lax.broadcasted_iota(jnp.int32, (TB, n), 1)
                    oh = (iota == ix).astype(jnp.bfloat16)
                    g = jnp.dot(oh, eks[off:off + n],
                                preferred_element_type=jnp.float32
                                ).astype(jnp.bfloat16)
                a = g[:, :D] + u
                b = g[:, D:] + iu
                r = (a * b) / (a + b)
                t = jnp.sum(r, axis=-1, keepdims=True
                            ).astype(jnp.float32)
                bit = t <= 1.0
                ix = 2 * ix + bit.astype(jnp.int32)
                acc = acc + jnp.minimum(t, 1.0 / t)
            iota = jax.lax.broadcasted_iota(jnp.int32, (TB, 1 << _DEPTH), 1)
            oh = (iota == ix).astype(jnp.float32)
            val_out_ref[:, h, :] = jnp.dot(oh, val_ref[h],
                                           preferred_element_type=jnp.float32)
        sup_ref[...] = -jnp.log(acc)


def kernel(query, tree_key, tree_value, random):
    B, D = query.shape
    H = tree_key.shape[0]
    out_dim = tree_value.shape[-1]
    nk = (1 << _DEPTH) - 1
    nv = 1 << _DEPTH
    TB = 256
    nb = B // TB

    sup, val = pl.pallas_call(
        _tree_kernel,
        grid=(H + nb,),
        in_specs=[
            pl.BlockSpec((1, nk, 2, D),
                         lambda i: (jnp.minimum(i, _H - 1), 0, 0, 0)),
            pl.BlockSpec((TB, D), lambda i: (jnp.maximum(i - _H, 0), 0)),
            pl.BlockSpec((H, nv, out_dim), lambda i: (0, 0, 0)),
        ],
        out_specs=[
            pl.BlockSpec((TB, 1), lambda i: (jnp.maximum(i - _H, 0), 0)),
            pl.BlockSpec((TB, H, out_dim),
                         lambda i: (jnp.maximum(i - _H, 0), 0, 0)),
        ],
        out_shape=[
            jax.ShapeDtypeStruct((B, 1), jnp.float32),
            jax.ShapeDtypeStruct((B, H, out_dim), jnp.float32),
        ],
        scratch_shapes=[pltpu.VMEM((H, _KROWS, 2 * D), jnp.bfloat16)],
    )(tree_key, query, tree_value)
    return sup.reshape(B), val.reshape(B * H, out_dim)


# speedup vs baseline: 1.0397x; 1.0037x over previous
import jax
import jax.numpy as jnp
from jax.experimental import pallas as pl
from jax.experimental.pallas import tpu as pltpu

_DEPTH = 8
_H = 8
_LVL_OFF = (0, 8, 16, 24, 32, 48, 80, 144)
_KROWS = 272


def _tree_kernel(k_ref, q_ref, val_ref, sup_ref, val_out_ref, ek_ref):
    i = pl.program_id(0)

    @pl.when(i < _H)
    def _prologue():
        for d in range(_DEPTH):
            n = 1 << d
            off = _LVL_OFF[d]
            blk = k_ref[0, n - 1:2 * n - 1]
            D = blk.shape[-1]
            ek_ref[i, off:off + n, :D] = (
                jnp.exp(-blk[:, 0, :]).astype(jnp.bfloat16))
            ek_ref[i, off:off + n, D:] = (
                jnp.exp(-blk[:, 1, :]).astype(jnp.bfloat16))

    @pl.when(i >= _H)
    def _main():
        q = q_ref[...]
        TB, D = q.shape
        u = jnp.exp(-q).astype(jnp.bfloat16)
        iu = jnp.exp(q).astype(jnp.bfloat16)
        acc = jnp.zeros((TB, 1), jnp.float32)
        for h in range(_H):
            eks = ek_ref[h]
            ix = jnp.zeros((TB, 1), jnp.int32)
            for d in range(_DEPTH):
                n = 1 << d
                off = _LVL_OFF[d]
                if n == 1:
                    g = jnp.broadcast_to(eks[off:off + 1], (TB, 2 * D))
                elif n == 2:
                    m = (ix == 1).astype(jnp.bfloat16)
                    m0 = (ix == 0).astype(jnp.bfloat16)
                    g = (m0 * eks[off:off + 1]
                         + m * eks[off + 1:off + 2])
                elif n == 4:
                    g = (ix == 0).astype(jnp.bfloat16) * eks[off:off + 1]
                    for j in range(1, 4):
                        g = g + ((ix == j).astype(jnp.bfloat16)
                                 * eks[off + j:off + j + 1])
                else:
                    iota = jax.lax.broadcasted_iota(jnp.int32, (TB, n), 1)
                    oh = (iota == ix).astype(jnp.bfloat16)
                    g = jnp.dot(oh, eks[off:off + n],
                                preferred_element_type=jnp.float32
                                ).astype(jnp.bfloat16)
                a = g[:, :D] + u
                b = g[:, D:] + iu
                r = (a * b) / (a + b)
                t = jnp.sum(r, axis=-1, keepdims=True
                            ).astype(jnp.float32)
                bit = t <= 1.0
                ix = 2 * ix + bit.astype(jnp.int32)
                acc = acc + jnp.minimum(t, 1.0 / t)
            iota = jax.lax.broadcasted_iota(jnp.int32, (TB, 1 << _DEPTH), 1)
            oh = (iota == ix).astype(jnp.float32)
            val_out_ref[:, h, :] = jnp.dot(oh, val_ref[h],
                                           preferred_element_type=jnp.float32)
        sup_ref[...] = -jnp.log(acc)


def kernel(query, tree_key, tree_value, random):
    B, D = query.shape
    H = tree_key.shape[0]
    out_dim = tree_value.shape[-1]
    nk = (1 << _DEPTH) - 1
    nv = 1 << _DEPTH
    TB = 256
    nb = B // TB

    sup, val = pl.pallas_call(
        _tree_kernel,
        grid=(H + nb,),
        in_specs=[
            pl.BlockSpec((1, nk, 2, D),
                         lambda i: (jnp.minimum(i, _H - 1), 0, 0, 0)),
            pl.BlockSpec((TB, D), lambda i: (jnp.maximum(i - _H, 0), 0)),
            pl.BlockSpec((H, nv, out_dim), lambda i: (0, 0, 0)),
        ],
        out_specs=[
            pl.BlockSpec((TB, 1), lambda i: (jnp.maximum(i - _H, 0), 0)),
            pl.BlockSpec((TB, H, out_dim),
                         lambda i: (jnp.maximum(i - _H, 0), 0, 0)),
        ],
        out_shape=[
            jax.ShapeDtypeStruct((B, 1), jnp.float32),
            jax.ShapeDtypeStruct((B, H, out_dim), jnp.float32),
        ],
        scratch_shapes=[pltpu.VMEM((H, _KROWS, 2 * D), jnp.bfloat16)],
    )(tree_key, query, tree_value)
    return sup.reshape(B), val.reshape(B * H, out_dim)
